# baseline (speedup 1.0000x reference)
import jax
import jax.numpy as jnp
from jax import lax
from jax.experimental import pallas as pl
from jax.experimental.pallas import tpu as pltpu
from jax.experimental.pallas import tpu_sc as plsc

_N_DENSE = 13
_N_SPARSE = 26
_VOCAB = 100000
_DIM = 32
_B = 16384
_ROW = _N_DENSE + _N_SPARSE * _DIM

_NC = 2
_NS = 16
_CHUNK = 4096
_NCHUNK = _B // _CHUNK
_SEGS = _CHUNK // 16
_UNROLL = 16


def _body(xt_hbm, tt_hbm, out_hbm, slab_v, idx_v, res_v, slab_sem,
          idx_sem, res_sem):
    d = lax.axis_index("s") * _NC + lax.axis_index("c")

    def idx_start(t, k, buf):
        return pltpu.async_copy(
            xt_hbm.at[_N_DENSE + t, pl.ds(k * _CHUNK, _CHUNK)],
            idx_v.at[buf], idx_sem)

    def slab_start(t):
        for p in range(4):
            lo = p * (_VOCAB // 4)
            pltpu.async_copy(tt_hbm.at[t, d].at[pl.ds(lo, _VOCAB // 4)],
                             slab_v.at[pl.ds(lo, _VOCAB // 4)], slab_sem)

    def slab_wait(t):
        for p in range(4):
            lo = p * (_VOCAB // 4)
            pltpu.make_async_copy(
                tt_hbm.at[t, d].at[pl.ds(lo, _VOCAB // 4)],
                slab_v.at[pl.ds(lo, _VOCAB // 4)], slab_sem).wait()

    slab_start(0)
    idx_start(0, 0, 0)
    idx_start(0, 1, 1)

    def field_body(t, carry):
        orow = _N_DENSE + t * _DIM + d
        slab_wait(t)

        for k in range(_NCHUNK):
            buf = k % 2
            pltpu.make_async_copy(
                xt_hbm.at[_N_DENSE + t, pl.ds(k * _CHUNK, _CHUNK)],
                idx_v.at[buf], idx_sem).wait()
            if k >= 2:
                pltpu.make_async_copy(
                    res_v.at[buf],
                    out_hbm.at[orow, pl.ds((k - 2) * _CHUNK, _CHUNK)],
                    res_sem).wait()
            else:
                @pl.when(t > 0)
                def _():
                    pltpu.make_async_copy(
                        res_v.at[buf],
                        out_hbm.at[orow - _DIM,
                                   pl.ds((_NCHUNK - 2 + k) * _CHUNK, _CHUNK)],
                        res_sem).wait()

            def seg_body(i, c2):
                s0 = i * _UNROLL
                for u in range(_UNROLL):
                    off = (s0 + u) * 16
                    iv = idx_v[buf, pl.ds(off, 16)]
                    ev = plsc.load_gather(slab_v, [iv.astype(jnp.int32)])
                    res_v[buf, pl.ds(off, 16)] = ev
                return c2

            lax.fori_loop(0, _SEGS // _UNROLL, seg_body, 0)

            if k == _NCHUNK - 1:
                @pl.when(t + 1 < _N_SPARSE)
                def _():
                    slab_start(t + 1)
            nk = k + 2
            if nk < _NCHUNK:
                idx_start(t, nk, nk % 2)
            else:
                @pl.when(t + 1 < _N_SPARSE)
                def _():
                    idx_start(t + 1, nk - _NCHUNK, nk % 2)
            pltpu.async_copy(
                res_v.at[buf],
                out_hbm.at[orow, pl.ds(k * _CHUNK, _CHUNK)], res_sem)
        return carry

    lax.fori_loop(0, _N_SPARSE, field_body, 0)

    last_row = _N_DENSE + (_N_SPARSE - 1) * _DIM + d
    for k in (_NCHUNK - 2, _NCHUNK - 1):
        pltpu.make_async_copy(
            res_v.at[k % 2],
            out_hbm.at[last_row, pl.ds(k * _CHUNK, _CHUNK)], res_sem).wait()

    @pl.when(d < _N_DENSE)
    def _():
        def dchunk(k, carry):
            pltpu.sync_copy(xt_hbm.at[d, pl.ds(k * _CHUNK, _CHUNK)],
                            res_v.at[0])
            pltpu.sync_copy(res_v.at[0],
                            out_hbm.at[d, pl.ds(k * _CHUNK, _CHUNK)])
            return carry

        lax.fori_loop(0, _NCHUNK, dchunk, 0)


@jax.jit
def kernel(x, tables):
    xt = x.T
    tt = jnp.transpose(tables, (0, 2, 1))
    mesh = plsc.VectorSubcoreMesh(core_axis_name="c", subcore_axis_name="s")
    out_t = pl.kernel(
        _body,
        mesh=mesh,
        compiler_params=pltpu.CompilerParams(
            needs_layout_passes=False, use_tc_tiling_on_sc=False
        ),
        out_type=jax.ShapeDtypeStruct((_ROW, _B), jnp.float32),
        scratch_types=[
            pltpu.VMEM((_VOCAB,), jnp.float32),
            pltpu.VMEM((2, _CHUNK), jnp.float32),
            pltpu.VMEM((2, _CHUNK), jnp.float32),
            pltpu.SemaphoreType.DMA,
            pltpu.SemaphoreType.DMA,
            pltpu.SemaphoreType.DMA,
        ],
    )(xt, tt)
    return out_t.T


# speedup vs baseline: 2.5586x; 2.5586x over previous
import jax
import jax.numpy as jnp
from jax import lax
from jax.experimental import pallas as pl
from jax.experimental.pallas import tpu as pltpu
from jax.experimental.pallas import tpu_sc as plsc

_N_DENSE = 13
_N_SPARSE = 26
_VOCAB = 100000
_DIM = 32
_B = 16384
_ROW = _N_DENSE + _N_SPARSE * _DIM

_NC = 2
_NS = 16
_CHUNK = 4096
_NCHUNK = _B // _CHUNK
_SEGS = _CHUNK // 16
_UNROLL = 16


def _body(xt_hbm, tt_hbm, out_hbm, slab_v, idx_v, res_v, slab_sem,
          idx_sem, res_sem):
    d = lax.axis_index("s") * _NC + lax.axis_index("c")

    def idx_start(t, k, buf):
        return pltpu.async_copy(
            xt_hbm.at[_N_DENSE + t, pl.ds(k * _CHUNK, _CHUNK)],
            idx_v.at[buf], idx_sem)

    def slab_start(t):
        return pltpu.async_copy(tt_hbm.at[t, d], slab_v, slab_sem)

    def slab_wait(t):
        pltpu.make_async_copy(tt_hbm.at[t, d], slab_v, slab_sem).wait()

    slab_start(0)
    idx_start(0, 0, 0)
    idx_start(0, 1, 1)

    def field_body(t, carry):
        orow = _N_DENSE + t * _DIM + d
        slab_wait(t)

        for k in range(_NCHUNK):
            buf = k % 2
            pltpu.make_async_copy(
                xt_hbm.at[_N_DENSE + t, pl.ds(k * _CHUNK, _CHUNK)],
                idx_v.at[buf], idx_sem).wait()
            if k >= 2:
                pltpu.make_async_copy(
                    res_v.at[buf],
                    out_hbm.at[orow, pl.ds((k - 2) * _CHUNK, _CHUNK)],
                    res_sem).wait()
            else:
                @pl.when(t > 0)
                def _():
                    pltpu.make_async_copy(
                        res_v.at[buf],
                        out_hbm.at[orow - _DIM,
                                   pl.ds((k + 2) * _CHUNK, _CHUNK)],
                        res_sem).wait()

            def seg_body(i, c2):
                s0 = i * _UNROLL
                for u in range(_UNROLL):
                    off = (s0 + u) * 16
                    iv = idx_v[buf, pl.ds(off, 16)]
                    ev = plsc.load_gather(slab_v, [iv.astype(jnp.int32)])
                    res_v[buf, pl.ds(off, 16)] = ev
                return c2

            lax.fori_loop(0, _SEGS // _UNROLL, seg_body, 0)

            if k == _NCHUNK - 1:
                @pl.when(t + 1 < _N_SPARSE)
                def _():
                    slab_start(t + 1)
            nk = k + 2
            if nk < _NCHUNK:
                idx_start(t, nk, nk % 2)
            else:
                @pl.when(t + 1 < _N_SPARSE)
                def _():
                    idx_start(t + 1, nk - _NCHUNK, nk % 2)
            pltpu.async_copy(
                res_v.at[buf],
                out_hbm.at[orow, pl.ds(k * _CHUNK, _CHUNK)], res_sem)
        return carry

    lax.fori_loop(0, _N_SPARSE, field_body, 0)

    last_row = _N_DENSE + (_N_SPARSE - 1) * _DIM + d
    for k in (_NCHUNK - 2, _NCHUNK - 1):
        pltpu.make_async_copy(
            res_v.at[k % 2],
            out_hbm.at[last_row, pl.ds(k * _CHUNK, _CHUNK)], res_sem).wait()

    @pl.when(d < _N_DENSE)
    def _():
        def dchunk(k, carry):
            pltpu.sync_copy(xt_hbm.at[d, pl.ds(k * _CHUNK, _CHUNK)],
                            res_v.at[0])
            pltpu.sync_copy(res_v.at[0],
                            out_hbm.at[d, pl.ds(k * _CHUNK, _CHUNK)])
            return carry

        lax.fori_loop(0, _NCHUNK, dchunk, 0)


@jax.jit
def kernel(x, tables):
    xt = x.T
    tt = jnp.transpose(tables, (0, 2, 1))
    mesh = plsc.VectorSubcoreMesh(core_axis_name="c", subcore_axis_name="s")
    out_t = pl.kernel(
        _body,
        mesh=mesh,
        compiler_params=pltpu.CompilerParams(
            needs_layout_passes=False, use_tc_tiling_on_sc=True
        ),
        out_type=jax.ShapeDtypeStruct((_ROW, _B), jnp.float32),
        scratch_types=[
            pltpu.VMEM((_VOCAB,), jnp.float32),
            pltpu.VMEM((2, _CHUNK), jnp.float32),
            pltpu.VMEM((2, _CHUNK), jnp.float32),
            pltpu.SemaphoreType.DMA,
            pltpu.SemaphoreType.DMA,
            pltpu.SemaphoreType.DMA,
        ],
    )(xt, tt)
    return out_t.T
